# baseline (speedup 1.0000x reference)
import functools

import jax
import jax.numpy as jnp
from jax.experimental import pallas as pl
from jax.experimental.pallas import tpu as pltpu


def _split3_bf16(v):
    hi = v.astype(jnp.bfloat16)
    r1 = v - hi.astype(jnp.float32)
    mid = r1.astype(jnp.bfloat16)
    lo = (r1 - mid.astype(jnp.float32)).astype(jnp.bfloat16)
    return hi, mid, lo


def _agg(ahat_bf16, h, dinv_col):
    f32 = jnp.float32
    tdot = functools.partial(
        jax.lax.dot_general,
        dimension_numbers=(((1,), (1,)), ((0,), (0,))),
        preferred_element_type=f32,
    )
    hi, mid, lo = _split3_bf16(h * dinv_col)
    acc = tdot(ahat_bf16, hi) + tdot(ahat_bf16, mid) + tdot(ahat_bf16, lo)
    return acc * dinv_col


def _gcn_group_kernel(x_ref, w1_ref, b1_ref, w2_ref, b2_ref,
                      wout_ref, bout_ref, out_ref, *, k, g):
    f32 = jnp.float32
    xg = x_ref[...]
    n = xg.shape[1]

    xwprec = jax.lax.Precision.DEFAULT
    xw1 = jax.lax.dot(
        xg.reshape(g * n, n), w1_ref[...],
        preferred_element_type=f32, precision=xwprec).reshape(g, n, -1)

    abits = jax.lax.bitcast_convert_type(xg, jnp.int32) & jnp.int32(0x7FFFFFFF)

    ones8 = jnp.ones((n, 8), f32)
    kf = f32(k)

    def body(t, prefix):
        bit = 30 - t
        cand = prefix | (jnp.int32(1) << bit)
        ge = abits >= cand.reshape(g, 1, 1)
        gf = jnp.where(ge, f32(1.0), f32(0.0))
        d1 = jax.lax.dot_general(gf, ones8, (((2,), (0,)), ((), ())),
                                 preferred_element_type=f32)
        cnt = jnp.sum(d1, axis=(1, 2)) * f32(0.125)
        return jnp.where(cnt >= kf, cand, prefix)

    thr = jax.lax.fori_loop(0, 31, body, jnp.zeros((g,), jnp.int32))
    thr3 = thr.reshape(g, 1, 1)

    gt = abits > thr3
    tie = abits == thr3
    need = jnp.int32(k) - jnp.sum(gt.astype(jnp.int32), axis=(1, 2))

    i0 = jax.lax.broadcasted_iota(jnp.int32, (n, n), 0)
    i1 = jax.lax.broadcasted_iota(jnp.int32, (n, n), 1)
    upper = (i0 < i1).astype(f32)
    eye = (i0 == i1).astype(f32)

    tie_f = tie.astype(f32)
    rank_in_row = jax.lax.dot(
        tie_f.reshape(g * n, n), upper,
        preferred_element_type=f32).reshape(g, n, n)
    row_tot = jnp.sum(tie_f, axis=2)
    row_off = jax.lax.dot(row_tot, upper, preferred_element_type=f32)
    rank = rank_in_row + row_off[:, :, None]
    keep_tie = jnp.logical_and(tie, rank < need.astype(f32).reshape(g, 1, 1))

    mask = jnp.where(jnp.logical_or(gt, keep_tie), f32(1.0), f32(0.0))
    ahat = mask + eye[None, :, :]
    deg = jnp.sum(ahat, axis=1)
    dinv = jax.lax.rsqrt(deg)

    hi = jax.lax.Precision.HIGHEST
    ahat_bf = ahat.astype(jnp.bfloat16)
    dinv_col = dinv[:, :, None]

    h1 = jnp.maximum(_agg(ahat_bf, xw1, dinv_col) + b1_ref[...], f32(0.0))

    h1w2 = jax.lax.dot(
        h1.reshape(g * n, -1), w2_ref[...],
        preferred_element_type=f32, precision=xwprec).reshape(g, n, -1)

    h2 = jnp.maximum(_agg(ahat_bf, h1w2, dinv_col) + b2_ref[...], f32(0.0))
    pooled = jnp.sum(h2, axis=1) * f32(1.0 / n)

    out_ref[...] = jax.lax.dot(pooled, wout_ref[...],
                               preferred_element_type=f32,
                               precision=hi) + bout_ref[...]


def kernel(x, adj, W1, b1, W2, b2, Wout, bout):
    del adj
    B, N, _ = x.shape
    k = int(N * N * 0.3)
    F1 = W1.shape[1]
    F2 = W2.shape[1]
    FO = Wout.shape[1]
    G = 8

    out = pl.pallas_call(
        functools.partial(_gcn_group_kernel, k=k, g=G),
        grid=(B // G,),
        in_specs=[
            pl.BlockSpec((G, N, N), lambda b: (b, 0, 0)),
            pl.BlockSpec((N, F1), lambda b: (0, 0)),
            pl.BlockSpec((1, F1), lambda b: (0, 0)),
            pl.BlockSpec((F1, F2), lambda b: (0, 0)),
            pl.BlockSpec((1, F2), lambda b: (0, 0)),
            pl.BlockSpec((F2, FO), lambda b: (0, 0)),
            pl.BlockSpec((1, FO), lambda b: (0, 0)),
        ],
        out_specs=pl.BlockSpec((G, FO), lambda b: (b, 0)),
        out_shape=jax.ShapeDtypeStruct((B, FO), jnp.float32),
        compiler_params=pltpu.CompilerParams(
            dimension_semantics=("parallel",)),
    )(x, W1, b1.reshape(1, F1), W2, b2.reshape(1, F2),
      Wout, bout.reshape(1, FO))
    return out


# speedup vs baseline: 1.7346x; 1.7346x over previous
import functools

import jax
import jax.numpy as jnp
from jax.experimental import pallas as pl
from jax.experimental.pallas import tpu as pltpu


def _split3_bf16(v):
    hi = v.astype(jnp.bfloat16)
    r1 = v - hi.astype(jnp.float32)
    mid = r1.astype(jnp.bfloat16)
    lo = (r1 - mid.astype(jnp.float32)).astype(jnp.bfloat16)
    return hi, mid, lo


def _agg(ahat_bf16, h, dinv_col):
    f32 = jnp.float32
    tdot = functools.partial(
        jax.lax.dot_general,
        dimension_numbers=(((1,), (1,)), ((0,), (0,))),
        preferred_element_type=f32,
    )
    hi, mid, lo = _split3_bf16(h * dinv_col)
    acc = tdot(ahat_bf16, hi) + tdot(ahat_bf16, mid) + tdot(ahat_bf16, lo)
    return acc * dinv_col


def _gcn_group_kernel(x_ref, w1_ref, b1_ref, w2_ref, b2_ref,
                      wout_ref, bout_ref, out_ref, *, k, g):
    f32 = jnp.float32
    xg = x_ref[...]
    n = xg.shape[1]

    xwprec = jax.lax.Precision.DEFAULT
    xw1 = jax.lax.dot(
        xg.reshape(g * n, n), w1_ref[...],
        preferred_element_type=f32, precision=xwprec).reshape(g, n, -1)

    abits = jax.lax.bitcast_convert_type(xg, jnp.int32) & jnp.int32(0x7FFFFFFF)

    def body(t, prefix):
        bit = 30 - t
        cand = prefix | (jnp.int32(1) << bit)
        ge = abits >= cand.reshape(g, 1, 1)
        cnt = jnp.sum(ge.astype(jnp.int32), axis=(1, 2))
        return jnp.where(cnt >= k, cand, prefix)

    thr = jax.lax.fori_loop(0, 31, body, jnp.zeros((g,), jnp.int32))
    thr3 = thr.reshape(g, 1, 1)

    gt = abits > thr3
    tie = abits == thr3
    need = jnp.int32(k) - jnp.sum(gt.astype(jnp.int32), axis=(1, 2))

    i0 = jax.lax.broadcasted_iota(jnp.int32, (n, n), 0)
    i1 = jax.lax.broadcasted_iota(jnp.int32, (n, n), 1)
    upper = (i0 < i1).astype(f32)
    eye = (i0 == i1).astype(f32)

    tie_f = tie.astype(f32)
    rank_in_row = jax.lax.dot(
        tie_f.reshape(g * n, n), upper,
        preferred_element_type=f32).reshape(g, n, n)
    row_tot = jnp.sum(tie_f, axis=2)
    row_off = jax.lax.dot(row_tot, upper, preferred_element_type=f32)
    rank = rank_in_row + row_off[:, :, None]
    keep_tie = jnp.logical_and(tie, rank < need.astype(f32).reshape(g, 1, 1))

    mask = jnp.where(jnp.logical_or(gt, keep_tie), f32(1.0), f32(0.0))
    ahat = mask + eye[None, :, :]
    deg = jnp.sum(ahat, axis=1)
    dinv = jax.lax.rsqrt(deg)

    hi = jax.lax.Precision.HIGHEST
    ahat_bf = ahat.astype(jnp.bfloat16)
    dinv_col = dinv[:, :, None]

    h1 = jnp.maximum(_agg(ahat_bf, xw1, dinv_col) + b1_ref[...], f32(0.0))

    h1w2 = jax.lax.dot(
        h1.reshape(g * n, -1), w2_ref[...],
        preferred_element_type=f32, precision=xwprec).reshape(g, n, -1)

    h2 = jnp.maximum(_agg(ahat_bf, h1w2, dinv_col) + b2_ref[...], f32(0.0))
    pooled = jnp.sum(h2, axis=1) * f32(1.0 / n)

    out_ref[...] = jax.lax.dot(pooled, wout_ref[...],
                               preferred_element_type=f32,
                               precision=hi) + bout_ref[...]


def kernel(x, adj, W1, b1, W2, b2, Wout, bout):
    del adj
    B, N, _ = x.shape
    k = int(N * N * 0.3)
    F1 = W1.shape[1]
    F2 = W2.shape[1]
    FO = Wout.shape[1]
    G = 16

    out = pl.pallas_call(
        functools.partial(_gcn_group_kernel, k=k, g=G),
        grid=(B // G,),
        in_specs=[
            pl.BlockSpec((G, N, N), lambda b: (b, 0, 0)),
            pl.BlockSpec((N, F1), lambda b: (0, 0)),
            pl.BlockSpec((1, F1), lambda b: (0, 0)),
            pl.BlockSpec((F1, F2), lambda b: (0, 0)),
            pl.BlockSpec((1, F2), lambda b: (0, 0)),
            pl.BlockSpec((F2, FO), lambda b: (0, 0)),
            pl.BlockSpec((1, FO), lambda b: (0, 0)),
        ],
        out_specs=pl.BlockSpec((G, FO), lambda b: (b, 0)),
        out_shape=jax.ShapeDtypeStruct((B, FO), jnp.float32),
        compiler_params=pltpu.CompilerParams(
            dimension_semantics=("parallel",)),
    )(x, W1, b1.reshape(1, F1), W2, b2.reshape(1, F2),
      Wout, bout.reshape(1, FO))
    return out


# speedup vs baseline: 1.8554x; 1.0696x over previous
import functools

import jax
import jax.numpy as jnp
from jax.experimental import pallas as pl
from jax.experimental.pallas import tpu as pltpu


def _split3_bf16(v):
    hi = v.astype(jnp.bfloat16)
    r1 = v - hi.astype(jnp.float32)
    mid = r1.astype(jnp.bfloat16)
    lo = (r1 - mid.astype(jnp.float32)).astype(jnp.bfloat16)
    return hi, mid, lo


def _agg(ahat_bf16, h, dinv_col):
    f32 = jnp.float32
    tdot = functools.partial(
        jax.lax.dot_general,
        dimension_numbers=(((1,), (1,)), ((0,), (0,))),
        preferred_element_type=f32,
    )
    hi, mid, lo = _split3_bf16(h * dinv_col)
    acc = tdot(ahat_bf16, hi) + tdot(ahat_bf16, mid) + tdot(ahat_bf16, lo)
    return acc * dinv_col


def _gcn_group_kernel(x_ref, w1_ref, b1_ref, w2_ref, b2_ref,
                      wout_ref, bout_ref, out_ref, *, k, g):
    f32 = jnp.float32
    xg = x_ref[...]
    n = xg.shape[1]

    xwprec = jax.lax.Precision.DEFAULT
    xw1 = jax.lax.dot(
        xg.reshape(g * n, n), w1_ref[...],
        preferred_element_type=f32, precision=xwprec).reshape(g, n, -1)

    abits = jax.lax.bitcast_convert_type(xg, jnp.int32) & jnp.int32(0x7FFFFFFF)

    def body(t, prefix):
        bit = 30 - t
        cand = prefix | (jnp.int32(1) << bit)
        ge = abits >= cand.reshape(g, 1, 1)
        cnt = jnp.sum(ge.astype(jnp.int32), axis=(1, 2))
        return jnp.where(cnt >= k, cand, prefix)

    thr = jax.lax.fori_loop(0, 31, body, jnp.zeros((g,), jnp.int32))
    thr3 = thr.reshape(g, 1, 1)

    gt = abits > thr3
    tie = abits == thr3
    need = jnp.int32(k) - jnp.sum(gt.astype(jnp.int32), axis=(1, 2))

    i0 = jax.lax.broadcasted_iota(jnp.int32, (n, n), 0)
    i1 = jax.lax.broadcasted_iota(jnp.int32, (n, n), 1)
    upper = (i0 < i1).astype(f32)
    eye = (i0 == i1).astype(f32)

    tie_f = tie.astype(f32)
    rank_in_row = jax.lax.dot(
        tie_f.reshape(g * n, n), upper,
        preferred_element_type=f32).reshape(g, n, n)
    row_tot = jnp.sum(tie_f, axis=2)
    row_off = jax.lax.dot(row_tot, upper, preferred_element_type=f32)
    rank = rank_in_row + row_off[:, :, None]
    keep_tie = jnp.logical_and(tie, rank < need.astype(f32).reshape(g, 1, 1))

    mask = jnp.where(jnp.logical_or(gt, keep_tie), f32(1.0), f32(0.0))
    ahat = mask + eye[None, :, :]
    deg = jnp.sum(ahat, axis=1)
    dinv = jax.lax.rsqrt(deg)

    hi = jax.lax.Precision.HIGHEST
    ahat_bf = ahat.astype(jnp.bfloat16)
    dinv_col = dinv[:, :, None]

    h1 = jnp.maximum(_agg(ahat_bf, xw1, dinv_col) + b1_ref[...], f32(0.0))

    h1w2 = jax.lax.dot(
        h1.reshape(g * n, -1), w2_ref[...],
        preferred_element_type=f32, precision=xwprec).reshape(g, n, -1)

    h2 = jnp.maximum(_agg(ahat_bf, h1w2, dinv_col) + b2_ref[...], f32(0.0))
    pooled = jnp.sum(h2, axis=1) * f32(1.0 / n)

    out_ref[...] = jax.lax.dot(pooled, wout_ref[...],
                               preferred_element_type=f32,
                               precision=hi) + bout_ref[...]


def kernel(x, adj, W1, b1, W2, b2, Wout, bout):
    del adj
    B, N, _ = x.shape
    k = int(N * N * 0.3)
    F1 = W1.shape[1]
    F2 = W2.shape[1]
    FO = Wout.shape[1]
    G = 32

    out = pl.pallas_call(
        functools.partial(_gcn_group_kernel, k=k, g=G),
        grid=(B // G,),
        in_specs=[
            pl.BlockSpec((G, N, N), lambda b: (b, 0, 0)),
            pl.BlockSpec((N, F1), lambda b: (0, 0)),
            pl.BlockSpec((1, F1), lambda b: (0, 0)),
            pl.BlockSpec((F1, F2), lambda b: (0, 0)),
            pl.BlockSpec((1, F2), lambda b: (0, 0)),
            pl.BlockSpec((F2, FO), lambda b: (0, 0)),
            pl.BlockSpec((1, FO), lambda b: (0, 0)),
        ],
        out_specs=pl.BlockSpec((G, FO), lambda b: (b, 0)),
        out_shape=jax.ShapeDtypeStruct((B, FO), jnp.float32),
        compiler_params=pltpu.CompilerParams(
            dimension_semantics=("parallel",)),
    )(x, W1, b1.reshape(1, F1), W2, b2.reshape(1, F2),
      Wout, bout.reshape(1, FO))
    return out


# speedup vs baseline: 1.9034x; 1.0258x over previous
import functools

import jax
import jax.numpy as jnp
from jax.experimental import pallas as pl
from jax.experimental.pallas import tpu as pltpu


def _split3_bf16(v):
    hi = v.astype(jnp.bfloat16)
    r1 = v - hi.astype(jnp.float32)
    mid = r1.astype(jnp.bfloat16)
    lo = (r1 - mid.astype(jnp.float32)).astype(jnp.bfloat16)
    return hi, mid, lo


def _agg(ahat_bf16, h, dinv_col):
    f32 = jnp.float32
    tdot = functools.partial(
        jax.lax.dot_general,
        dimension_numbers=(((1,), (1,)), ((0,), (0,))),
        preferred_element_type=f32,
    )
    hi, mid, lo = _split3_bf16(h * dinv_col)
    acc = tdot(ahat_bf16, hi) + tdot(ahat_bf16, mid) + tdot(ahat_bf16, lo)
    return acc * dinv_col


def _gcn_group_kernel(x_ref, w1_ref, b1_ref, w2_ref, b2_ref,
                      wout_ref, bout_ref, out_ref, *, k, g):
    f32 = jnp.float32
    xg = x_ref[...]
    n = xg.shape[1]

    xwprec = jax.lax.Precision.DEFAULT
    xw1 = jax.lax.dot(
        xg.reshape(g * n, n), w1_ref[...],
        preferred_element_type=f32, precision=xwprec).reshape(g, n, -1)

    abits = jax.lax.bitcast_convert_type(xg, jnp.int32) & jnp.int32(0x7FFFFFFF)

    def body(t, prefix):
        bit = 30 - t
        cand = prefix | (jnp.int32(1) << bit)
        ge = abits >= cand.reshape(g, 1, 1)
        cnt = jnp.sum(ge.astype(jnp.int32), axis=(1, 2))
        return jnp.where(cnt >= k, cand, prefix)

    thr = jax.lax.fori_loop(0, 31, body, jnp.zeros((g,), jnp.int32))
    thr3 = thr.reshape(g, 1, 1)

    gt = abits > thr3
    tie = abits == thr3
    need = jnp.int32(k) - jnp.sum(gt.astype(jnp.int32), axis=(1, 2))

    i0 = jax.lax.broadcasted_iota(jnp.int32, (n, n), 0)
    i1 = jax.lax.broadcasted_iota(jnp.int32, (n, n), 1)
    upper = (i0 < i1).astype(f32)
    eye = (i0 == i1).astype(f32)

    tie_f = tie.astype(f32)
    rank_in_row = jax.lax.dot(
        tie_f.reshape(g * n, n), upper,
        preferred_element_type=f32).reshape(g, n, n)
    row_tot = jnp.sum(tie_f, axis=2)
    row_off = jax.lax.dot(row_tot, upper, preferred_element_type=f32)
    rank = rank_in_row + row_off[:, :, None]
    keep_tie = jnp.logical_and(tie, rank < need.astype(f32).reshape(g, 1, 1))

    mask = jnp.where(jnp.logical_or(gt, keep_tie), f32(1.0), f32(0.0))
    ahat = mask + eye[None, :, :]
    deg = jnp.sum(ahat, axis=1)
    dinv = jax.lax.rsqrt(deg)

    hi = jax.lax.Precision.HIGHEST
    ahat_bf = ahat.astype(jnp.bfloat16)
    dinv_col = dinv[:, :, None]

    h1 = jnp.maximum(_agg(ahat_bf, xw1, dinv_col) + b1_ref[...], f32(0.0))

    h1w2 = jax.lax.dot(
        h1.reshape(g * n, -1), w2_ref[...],
        preferred_element_type=f32, precision=xwprec).reshape(g, n, -1)

    h2 = jnp.maximum(_agg(ahat_bf, h1w2, dinv_col) + b2_ref[...], f32(0.0))
    pooled = jnp.sum(h2, axis=1) * f32(1.0 / n)

    out_ref[...] = jax.lax.dot(pooled, wout_ref[...],
                               preferred_element_type=f32,
                               precision=hi) + bout_ref[...]


def kernel(x, adj, W1, b1, W2, b2, Wout, bout):
    del adj
    B, N, _ = x.shape
    k = int(N * N * 0.3)
    F1 = W1.shape[1]
    F2 = W2.shape[1]
    FO = Wout.shape[1]
    G = 64

    out = pl.pallas_call(
        functools.partial(_gcn_group_kernel, k=k, g=G),
        grid=(B // G,),
        in_specs=[
            pl.BlockSpec((G, N, N), lambda b: (b, 0, 0)),
            pl.BlockSpec((N, F1), lambda b: (0, 0)),
            pl.BlockSpec((1, F1), lambda b: (0, 0)),
            pl.BlockSpec((F1, F2), lambda b: (0, 0)),
            pl.BlockSpec((1, F2), lambda b: (0, 0)),
            pl.BlockSpec((F2, FO), lambda b: (0, 0)),
            pl.BlockSpec((1, FO), lambda b: (0, 0)),
        ],
        out_specs=pl.BlockSpec((G, FO), lambda b: (b, 0)),
        out_shape=jax.ShapeDtypeStruct((B, FO), jnp.float32),
        compiler_params=pltpu.CompilerParams(
            dimension_semantics=("parallel",)),
    )(x, W1, b1.reshape(1, F1), W2, b2.reshape(1, F2),
      Wout, bout.reshape(1, FO))
    return out


# speedup vs baseline: 1.9866x; 1.0437x over previous
import functools

import jax
import jax.numpy as jnp
from jax.experimental import pallas as pl
from jax.experimental.pallas import tpu as pltpu


def _split3_bf16(v):
    hi = v.astype(jnp.bfloat16)
    r1 = v - hi.astype(jnp.float32)
    mid = r1.astype(jnp.bfloat16)
    lo = (r1 - mid.astype(jnp.float32)).astype(jnp.bfloat16)
    return hi, mid, lo


def _agg(ahat_bf16, h, dinv_col):
    f32 = jnp.float32
    tdot = functools.partial(
        jax.lax.dot_general,
        dimension_numbers=(((1,), (1,)), ((0,), (0,))),
        preferred_element_type=f32,
    )
    hs = h * dinv_col
    hi = hs.astype(jnp.bfloat16)
    mid = (hs - hi.astype(f32)).astype(jnp.bfloat16)
    acc = tdot(ahat_bf16, hi) + tdot(ahat_bf16, mid)
    return acc * dinv_col


def _gcn_group_kernel(x_ref, w1_ref, b1_ref, w2_ref, b2_ref,
                      wout_ref, bout_ref, out_ref, *, k, g):
    f32 = jnp.float32
    xg = x_ref[...]
    n = xg.shape[1]

    xwprec = jax.lax.Precision.DEFAULT
    xw1 = jax.lax.dot(
        xg.reshape(g * n, n), w1_ref[...],
        preferred_element_type=f32, precision=xwprec).reshape(g, n, -1)

    abits = jax.lax.bitcast_convert_type(xg, jnp.int32) & jnp.int32(0x7FFFFFFF)

    def body(t, prefix):
        bit = 30 - t
        cand = prefix | (jnp.int32(1) << bit)
        ge = abits >= cand.reshape(g, 1, 1)
        cnt = jnp.sum(ge.astype(jnp.int32), axis=(1, 2))
        return jnp.where(cnt >= k, cand, prefix)

    thr = jax.lax.fori_loop(0, 31, body, jnp.zeros((g,), jnp.int32))
    thr3 = thr.reshape(g, 1, 1)

    gt = abits > thr3
    tie = abits == thr3
    need = jnp.int32(k) - jnp.sum(gt.astype(jnp.int32), axis=(1, 2))

    i0 = jax.lax.broadcasted_iota(jnp.int32, (n, n), 0)
    i1 = jax.lax.broadcasted_iota(jnp.int32, (n, n), 1)
    upper = (i0 < i1).astype(f32)
    eye = (i0 == i1).astype(f32)

    tie_f = tie.astype(f32)
    rank_in_row = jax.lax.dot(
        tie_f.reshape(g * n, n), upper,
        preferred_element_type=f32).reshape(g, n, n)
    row_tot = jnp.sum(tie_f, axis=2)
    row_off = jax.lax.dot(row_tot, upper, preferred_element_type=f32)
    rank = rank_in_row + row_off[:, :, None]
    keep_tie = jnp.logical_and(tie, rank < need.astype(f32).reshape(g, 1, 1))

    mask = jnp.where(jnp.logical_or(gt, keep_tie), f32(1.0), f32(0.0))
    ahat = mask + eye[None, :, :]
    deg = jnp.sum(ahat, axis=1)
    dinv = jax.lax.rsqrt(deg)

    hi = jax.lax.Precision.HIGHEST
    ahat_bf = ahat.astype(jnp.bfloat16)
    dinv_col = dinv[:, :, None]

    h1 = jnp.maximum(_agg(ahat_bf, xw1, dinv_col) + b1_ref[...], f32(0.0))

    h1w2 = jax.lax.dot(
        h1.reshape(g * n, -1), w2_ref[...],
        preferred_element_type=f32, precision=xwprec).reshape(g, n, -1)

    h2 = jnp.maximum(_agg(ahat_bf, h1w2, dinv_col) + b2_ref[...], f32(0.0))
    pooled = jnp.sum(h2, axis=1) * f32(1.0 / n)

    out_ref[...] = jax.lax.dot(pooled, wout_ref[...],
                               preferred_element_type=f32,
                               precision=hi) + bout_ref[...]


def kernel(x, adj, W1, b1, W2, b2, Wout, bout):
    del adj
    B, N, _ = x.shape
    k = int(N * N * 0.3)
    F1 = W1.shape[1]
    F2 = W2.shape[1]
    FO = Wout.shape[1]
    G = 64

    out = pl.pallas_call(
        functools.partial(_gcn_group_kernel, k=k, g=G),
        grid=(B // G,),
        in_specs=[
            pl.BlockSpec((G, N, N), lambda b: (b, 0, 0)),
            pl.BlockSpec((N, F1), lambda b: (0, 0)),
            pl.BlockSpec((1, F1), lambda b: (0, 0)),
            pl.BlockSpec((F1, F2), lambda b: (0, 0)),
            pl.BlockSpec((1, F2), lambda b: (0, 0)),
            pl.BlockSpec((F2, FO), lambda b: (0, 0)),
            pl.BlockSpec((1, FO), lambda b: (0, 0)),
        ],
        out_specs=pl.BlockSpec((G, FO), lambda b: (b, 0)),
        out_shape=jax.ShapeDtypeStruct((B, FO), jnp.float32),
        compiler_params=pltpu.CompilerParams(
            dimension_semantics=("parallel",)),
    )(x, W1, b1.reshape(1, F1), W2, b2.reshape(1, F2),
      Wout, bout.reshape(1, FO))
    return out
